# baseline (speedup 1.0000x reference)
import functools

import jax
import jax.numpy as jnp
from jax import lax
from jax.experimental import pallas as pl
from jax.experimental.pallas import tpu as pltpu
from jax.experimental.pallas import tpu_sc as plsc

N = 100000
D = 128
H = 64
NUM_SEG = 2048

NTILES = 16
CHUNK = 6400
TAIL = N - (NTILES - 1) * CHUNK
SEG_SLICE = NUM_SEG // NTILES
L = 16


_BLK = 20000


def _mlp_body(x_ref, w1_ref, b1_ref, w2_ref, b2_ref, out_ref):
    h = jnp.dot(x_ref[...], w1_ref[...], preferred_element_type=jnp.float32)
    h = h + b1_ref[...]
    h = h * jax.nn.sigmoid(h)
    out_ref[...] = jnp.dot(h, w2_ref[...],
                           preferred_element_type=jnp.float32) + b2_ref[0]


def _mlp(x, w1, b1, w2col, b2):
    grid = (N // _BLK,)
    return pl.pallas_call(
        _mlp_body,
        grid=grid,
        in_specs=[
            pl.BlockSpec((_BLK, D), lambda i: (i, 0)),
            pl.BlockSpec((D, H), lambda i: (0, 0)),
            pl.BlockSpec((H,), lambda i: (0,)),
            pl.BlockSpec((H, 1), lambda i: (0, 0)),
            pl.BlockSpec((1,), lambda i: (0,)),
        ],
        out_specs=pl.BlockSpec((_BLK, 1), lambda i: (i, 0)),
        out_shape=jax.ShapeDtypeStruct((N, 1), jnp.float32),
        compiler_params=pltpu.CompilerParams(
            dimension_semantics=("arbitrary",),
        ),
    )(x, w1, b1, w2col, b2).reshape(N)


@functools.cache
def _build_sc_conserve():
    mesh = plsc.VectorSubcoreMesh(
        core_axis_name="c", subcore_axis_name="s", num_cores=1
    )
    return functools.partial(
        pl.kernel,
        out_type=jax.ShapeDtypeStruct((N,), jnp.float32),
        mesh=mesh,
        scratch_types=[
            pltpu.VMEM((CHUNK,), jnp.int32),
            pltpu.VMEM((CHUNK,), jnp.float32),
            pltpu.VMEM((CHUNK,), jnp.float32),
            pltpu.VMEM((CHUNK,), jnp.float32),
            pltpu.VMEM((NUM_SEG,), jnp.float32),
            pltpu.VMEM((SEG_SLICE,), jnp.float32),
            pltpu.VMEM((SEG_SLICE,), jnp.float32),
            pltpu.VMEM_SHARED((NUM_SEG,), jnp.float32),
            pltpu.VMEM_SHARED((NUM_SEG,), jnp.float32),
            pltpu.VMEM_SHARED((NUM_SEG,), jnp.float32),
            pltpu.SemaphoreType.DMA,
            pltpu.SemaphoreType.DMA,
            pltpu.SemaphoreType.DMA,
        ],
        compiler_params=pltpu.CompilerParams(needs_layout_passes=False),
    )(_sc_conserve_body)


def _sc_conserve_body(batch_hbm, charges_hbm, mask_hbm, out_hbm,
                      bvm, cvm, mvm, ovm, dvm, sa, sb,
                      raw_sh, cnt_sh, delta_sh, sem0, sem1, sem2):
    sid = lax.axis_index("s")
    base = sid * CHUNK
    seg_base = sid * SEG_SLICE
    is_tail = sid == NTILES - 1

    cm = pltpu.async_copy(mask_hbm.at[pl.ds(base, CHUNK)], mvm, sem2)

    @pl.when(jnp.logical_not(is_tail))
    def _full_loads():
        pltpu.async_copy(batch_hbm.at[pl.ds(base, CHUNK)], bvm, sem0)
        pltpu.async_copy(charges_hbm.at[pl.ds(base, CHUNK)], cvm, sem1)

    @pl.when(is_tail)
    def _tail_loads():
        pltpu.async_copy(batch_hbm.at[pl.ds(base, TAIL)],
                         bvm.at[pl.ds(0, TAIL)], sem0)
        pltpu.async_copy(charges_hbm.at[pl.ds(base, TAIL)],
                         cvm.at[pl.ds(0, TAIL)], sem1)

    def _zbody(i, _):
        sa[pl.ds(i * L, L)] = jnp.zeros((L,), jnp.float32)
        return 0
    lax.fori_loop(0, SEG_SLICE // L, _zbody, 0)
    pltpu.sync_copy(sa, raw_sh.at[pl.ds(seg_base, SEG_SLICE)])
    pltpu.sync_copy(sa, cnt_sh.at[pl.ds(seg_base, SEG_SLICE)])

    @pl.when(jnp.logical_not(is_tail))
    def _full_wait():
        pltpu.make_async_copy(batch_hbm.at[pl.ds(base, CHUNK)], bvm, sem0).wait()
        pltpu.make_async_copy(charges_hbm.at[pl.ds(base, CHUNK)], cvm, sem1).wait()

    @pl.when(is_tail)
    def _tail_wait():
        pltpu.make_async_copy(batch_hbm.at[pl.ds(base, TAIL)],
                              bvm.at[pl.ds(0, TAIL)], sem0).wait()
        pltpu.make_async_copy(charges_hbm.at[pl.ds(base, TAIL)],
                              cvm.at[pl.ds(0, TAIL)], sem1).wait()
        def _fbody(i, _):
            sl = pl.ds(TAIL + i * L, L)
            bvm[sl] = jnp.zeros((L,), jnp.int32)
            cvm[sl] = jnp.zeros((L,), jnp.float32)
            return 0
        lax.fori_loop(0, (CHUNK - TAIL) // L, _fbody, 0)

    cm.wait()
    plsc.subcore_barrier()

    pltpu.sync_copy(cvm, raw_sh.at[bvm], add=True)
    pltpu.sync_copy(mvm, cnt_sh.at[bvm], add=True)
    plsc.subcore_barrier()

    pltpu.sync_copy(raw_sh.at[pl.ds(seg_base, SEG_SLICE)], sa)
    pltpu.sync_copy(cnt_sh.at[pl.ds(seg_base, SEG_SLICE)], sb)

    def _dbody(i, _):
        sl = pl.ds(i * L, L)
        sa[sl] = (jnp.zeros((L,), jnp.float32) - sa[sl]) / jnp.maximum(
            sb[sl], jnp.ones((L,), jnp.float32))
        return 0
    lax.fori_loop(0, SEG_SLICE // L, _dbody, 0)
    pltpu.sync_copy(sa, delta_sh.at[pl.ds(seg_base, SEG_SLICE)])
    plsc.subcore_barrier()

    pltpu.sync_copy(delta_sh, dvm)

    def _gbody(i, _):
        sl = pl.ds(i * L, L)
        idx = bvm[sl]
        ovm[sl] = cvm[sl] + plsc.load_gather(dvm, [idx])
        return 0
    lax.fori_loop(0, CHUNK // L, _gbody, 0)

    @pl.when(jnp.logical_not(is_tail))
    def _full_store():
        pltpu.sync_copy(ovm, out_hbm.at[pl.ds(base, CHUNK)])

    @pl.when(is_tail)
    def _tail_store():
        pltpu.sync_copy(ovm.at[pl.ds(0, TAIL)], out_hbm.at[pl.ds(base, TAIL)])


def kernel(node_invariant, batch, W1, b1, W2, b2):
    charges = _mlp(node_invariant, W1, b1, W2, b2)
    mask = (jnp.arange(NTILES * CHUNK, dtype=jnp.int32) < N).astype(jnp.float32)
    return _build_sc_conserve()(batch, charges, mask)


# speedup vs baseline: 1.5030x; 1.5030x over previous
import functools

import jax
import jax.numpy as jnp
from jax import lax
from jax.experimental import pallas as pl
from jax.experimental.pallas import tpu as pltpu
from jax.experimental.pallas import tpu_sc as plsc

N = 100000
D = 128
H = 64
NUM_SEG = 2048

NTILES = 16
CHUNK = 6400
TAIL = N - (NTILES - 1) * CHUNK
SEG_SLICE = NUM_SEG // NTILES
L = 16


_BLK = 20000


def _mlp_body(x_ref, w1_ref, b1_ref, w2_ref, b2_ref, out_ref):
    ht = lax.dot_general(w1_ref[...], x_ref[...],
                         dimension_numbers=(((0,), (1,)), ((), ())),
                         preferred_element_type=jnp.float32)
    ht = ht + b1_ref[...].reshape(H, 1)
    ht = ht * jax.nn.sigmoid(ht)
    row = lax.dot_general(w2_ref[...], ht,
                          dimension_numbers=(((0,), (0,)), ((), ())),
                          preferred_element_type=jnp.float32) + b2_ref[0]
    out_ref[...] = row.reshape(1, 1, _BLK)


def _mlp(x, w1, b1, w2col, b2):
    grid = (N // _BLK,)
    return pl.pallas_call(
        _mlp_body,
        grid=grid,
        in_specs=[
            pl.BlockSpec((_BLK, D), lambda i: (i, 0)),
            pl.BlockSpec((D, H), lambda i: (0, 0)),
            pl.BlockSpec((H,), lambda i: (0,)),
            pl.BlockSpec((H, 1), lambda i: (0, 0)),
            pl.BlockSpec((1,), lambda i: (0,)),
        ],
        out_specs=pl.BlockSpec((1, 1, _BLK), lambda i: (i, 0, 0)),
        out_shape=jax.ShapeDtypeStruct((N // _BLK, 1, _BLK), jnp.float32),
        compiler_params=pltpu.CompilerParams(
            dimension_semantics=("arbitrary",),
        ),
    )(x, w1, b1, w2col, b2).reshape(N)


@functools.cache
def _build_sc_conserve():
    mesh = plsc.VectorSubcoreMesh(
        core_axis_name="c", subcore_axis_name="s", num_cores=1
    )
    return functools.partial(
        pl.kernel,
        out_type=jax.ShapeDtypeStruct((N,), jnp.float32),
        mesh=mesh,
        scratch_types=[
            pltpu.VMEM((CHUNK,), jnp.int32),
            pltpu.VMEM((CHUNK,), jnp.float32),
            pltpu.VMEM((CHUNK,), jnp.float32),
            pltpu.VMEM((CHUNK,), jnp.float32),
            pltpu.VMEM((NUM_SEG,), jnp.float32),
            pltpu.VMEM((SEG_SLICE,), jnp.float32),
            pltpu.VMEM((SEG_SLICE,), jnp.float32),
            pltpu.VMEM_SHARED((NUM_SEG,), jnp.float32),
            pltpu.VMEM_SHARED((NUM_SEG,), jnp.float32),
            pltpu.VMEM_SHARED((NUM_SEG,), jnp.float32),
            pltpu.SemaphoreType.DMA,
            pltpu.SemaphoreType.DMA,
            pltpu.SemaphoreType.DMA,
        ],
        compiler_params=pltpu.CompilerParams(needs_layout_passes=False),
    )(_sc_conserve_body)


def _sc_conserve_body(batch_hbm, charges_hbm, mask_hbm, out_hbm,
                      bvm, cvm, mvm, ovm, dvm, sa, sb,
                      raw_sh, cnt_sh, delta_sh, sem0, sem1, sem2):
    sid = lax.axis_index("s")
    base = sid * CHUNK
    seg_base = sid * SEG_SLICE
    is_tail = sid == NTILES - 1

    cm = pltpu.async_copy(mask_hbm.at[pl.ds(base, CHUNK)], mvm, sem2)

    @pl.when(jnp.logical_not(is_tail))
    def _full_loads():
        pltpu.async_copy(batch_hbm.at[pl.ds(base, CHUNK)], bvm, sem0)
        pltpu.async_copy(charges_hbm.at[pl.ds(base, CHUNK)], cvm, sem1)

    @pl.when(is_tail)
    def _tail_loads():
        pltpu.async_copy(batch_hbm.at[pl.ds(base, TAIL)],
                         bvm.at[pl.ds(0, TAIL)], sem0)
        pltpu.async_copy(charges_hbm.at[pl.ds(base, TAIL)],
                         cvm.at[pl.ds(0, TAIL)], sem1)

    def _zbody(i, _):
        sa[pl.ds(i * L, L)] = jnp.zeros((L,), jnp.float32)
        return 0
    lax.fori_loop(0, SEG_SLICE // L, _zbody, 0)
    pltpu.sync_copy(sa, raw_sh.at[pl.ds(seg_base, SEG_SLICE)])
    pltpu.sync_copy(sa, cnt_sh.at[pl.ds(seg_base, SEG_SLICE)])

    @pl.when(jnp.logical_not(is_tail))
    def _full_wait():
        pltpu.make_async_copy(batch_hbm.at[pl.ds(base, CHUNK)], bvm, sem0).wait()
        pltpu.make_async_copy(charges_hbm.at[pl.ds(base, CHUNK)], cvm, sem1).wait()

    @pl.when(is_tail)
    def _tail_wait():
        pltpu.make_async_copy(batch_hbm.at[pl.ds(base, TAIL)],
                              bvm.at[pl.ds(0, TAIL)], sem0).wait()
        pltpu.make_async_copy(charges_hbm.at[pl.ds(base, TAIL)],
                              cvm.at[pl.ds(0, TAIL)], sem1).wait()
        def _fbody(i, _):
            sl = pl.ds(TAIL + i * L, L)
            bvm[sl] = jnp.zeros((L,), jnp.int32)
            cvm[sl] = jnp.zeros((L,), jnp.float32)
            return 0
        lax.fori_loop(0, (CHUNK - TAIL) // L, _fbody, 0)

    cm.wait()
    plsc.subcore_barrier()

    pltpu.sync_copy(cvm, raw_sh.at[bvm], add=True)
    pltpu.sync_copy(mvm, cnt_sh.at[bvm], add=True)
    plsc.subcore_barrier()

    pltpu.sync_copy(raw_sh.at[pl.ds(seg_base, SEG_SLICE)], sa)
    pltpu.sync_copy(cnt_sh.at[pl.ds(seg_base, SEG_SLICE)], sb)

    def _dbody(i, _):
        sl = pl.ds(i * L, L)
        sa[sl] = (jnp.zeros((L,), jnp.float32) - sa[sl]) / jnp.maximum(
            sb[sl], jnp.ones((L,), jnp.float32))
        return 0
    lax.fori_loop(0, SEG_SLICE // L, _dbody, 0)
    pltpu.sync_copy(sa, delta_sh.at[pl.ds(seg_base, SEG_SLICE)])
    plsc.subcore_barrier()

    pltpu.sync_copy(delta_sh, dvm)

    def _gbody(i, _):
        sl = pl.ds(i * L, L)
        idx = bvm[sl]
        ovm[sl] = cvm[sl] + plsc.load_gather(dvm, [idx])
        return 0
    lax.fori_loop(0, CHUNK // L, _gbody, 0)

    @pl.when(jnp.logical_not(is_tail))
    def _full_store():
        pltpu.sync_copy(ovm, out_hbm.at[pl.ds(base, CHUNK)])

    @pl.when(is_tail)
    def _tail_store():
        pltpu.sync_copy(ovm.at[pl.ds(0, TAIL)], out_hbm.at[pl.ds(base, TAIL)])


def kernel(node_invariant, batch, W1, b1, W2, b2):
    charges = _mlp(node_invariant, W1, b1, W2, b2)
    mask = (jnp.arange(NTILES * CHUNK, dtype=jnp.int32) < N).astype(jnp.float32)
    return _build_sc_conserve()(batch, charges, mask)


# speedup vs baseline: 1.5776x; 1.0497x over previous
import functools

import jax
import jax.numpy as jnp
from jax import lax
from jax.experimental import pallas as pl
from jax.experimental.pallas import tpu as pltpu
from jax.experimental.pallas import tpu_sc as plsc

N = 100000
D = 128
H = 64
NUM_SEG = 2048

NTILES = 16
CHUNK = 6400
TAIL = N - (NTILES - 1) * CHUNK
SEG_SLICE = NUM_SEG // NTILES
L = 16


_BLK = 20000


def _mlp_body(x_ref, w1_ref, b1_ref, w2_ref, b2_ref, out_ref):
    ht = lax.dot_general(w1_ref[...], x_ref[...],
                         dimension_numbers=(((0,), (1,)), ((), ())),
                         preferred_element_type=jnp.float32)
    ht = ht + b1_ref[...].reshape(H, 1)
    ht = ht * jax.nn.sigmoid(ht)
    row = lax.dot_general(w2_ref[...], ht,
                          dimension_numbers=(((0,), (0,)), ((), ())),
                          preferred_element_type=jnp.float32) + b2_ref[0]
    out_ref[...] = row.reshape(1, 1, _BLK)


def _mlp(x, w1, b1, w2col, b2):
    grid = (N // _BLK,)
    return pl.pallas_call(
        _mlp_body,
        grid=grid,
        in_specs=[
            pl.BlockSpec((_BLK, D), lambda i: (i, 0)),
            pl.BlockSpec((D, H), lambda i: (0, 0)),
            pl.BlockSpec((H,), lambda i: (0,)),
            pl.BlockSpec((H, 1), lambda i: (0, 0)),
            pl.BlockSpec((1,), lambda i: (0,)),
        ],
        out_specs=pl.BlockSpec((1, 1, _BLK), lambda i: (i, 0, 0)),
        out_shape=jax.ShapeDtypeStruct((N // _BLK, 1, _BLK), jnp.float32),
        compiler_params=pltpu.CompilerParams(
            dimension_semantics=("arbitrary",),
        ),
    )(x, w1, b1, w2col, b2).reshape(N)


@functools.cache
def _build_sc_counts():
    mesh = plsc.VectorSubcoreMesh(
        core_axis_name="c", subcore_axis_name="s", num_cores=1
    )
    return functools.partial(
        pl.kernel,
        out_type=jax.ShapeDtypeStruct((NUM_SEG,), jnp.float32),
        mesh=mesh,
        scratch_types=[
            pltpu.VMEM((CHUNK,), jnp.int32),
            pltpu.VMEM((CHUNK,), jnp.float32),
            pltpu.VMEM((CHUNK,), jnp.int32),
            pltpu.VMEM((NUM_SEG,), jnp.float32),
            pltpu.VMEM((NTILES, SEG_SLICE), jnp.float32),
            pltpu.VMEM((SEG_SLICE,), jnp.float32),
            pltpu.VMEM_SHARED((NTILES * NUM_SEG,), jnp.float32),
            pltpu.SemaphoreType.DMA,
            pltpu.SemaphoreType.DMA,
        ],
        compiler_params=pltpu.CompilerParams(needs_layout_passes=False),
    )(_sc_counts_body)


def _sc_counts_body(batch_hbm, mask_hbm, inv_hbm,
                    bvm, mvm, ivm, tvm, rvm, sa, tab_sh, sem0, sem1):
    sid = lax.axis_index("s")
    base = sid * CHUNK
    seg_base = sid * SEG_SLICE
    is_tail = sid == NTILES - 1

    cm = pltpu.async_copy(mask_hbm.at[pl.ds(base, CHUNK)], mvm, sem1)

    @pl.when(jnp.logical_not(is_tail))
    def _full_loads():
        pltpu.async_copy(batch_hbm.at[pl.ds(base, CHUNK)], bvm, sem0)

    @pl.when(is_tail)
    def _tail_loads():
        pltpu.async_copy(batch_hbm.at[pl.ds(base, TAIL)],
                         bvm.at[pl.ds(0, TAIL)], sem0)

    def _zbody(i, _):
        tvm[pl.ds(i * L, L)] = jnp.zeros((L,), jnp.float32)
        return 0
    lax.fori_loop(0, NUM_SEG // L, _zbody, 0)

    @pl.when(jnp.logical_not(is_tail))
    def _full_wait():
        pltpu.make_async_copy(batch_hbm.at[pl.ds(base, CHUNK)], bvm, sem0).wait()

    @pl.when(is_tail)
    def _tail_wait():
        pltpu.make_async_copy(batch_hbm.at[pl.ds(base, TAIL)],
                              bvm.at[pl.ds(0, TAIL)], sem0).wait()
        def _fbody(i, _):
            bvm[pl.ds(TAIL + i * L, L)] = jnp.zeros((L,), jnp.int32)
            return 0
        lax.fori_loop(0, (CHUNK - TAIL) // L, _fbody, 0)

    cm.wait()

    pltpu.sync_copy(tvm, tab_sh.at[pl.ds(sid * NUM_SEG, NUM_SEG)])

    def _obody(i, _):
        sl = pl.ds(i * L, L)
        ivm[sl] = bvm[sl] + sid * NUM_SEG
        return 0
    lax.fori_loop(0, CHUNK // L, _obody, 0)
    pltpu.sync_copy(mvm, tab_sh.at[ivm], add=True)
    plsc.subcore_barrier()

    for t in range(NTILES):
        pltpu.sync_copy(
            tab_sh.at[pl.ds(t * NUM_SEG + seg_base, SEG_SLICE)], rvm.at[t])

    for s in range(SEG_SLICE // L):
        sl = pl.ds(s * L, L)
        acc = rvm[0, sl]
        for t in range(1, NTILES):
            acc = acc + rvm[t, sl]
        sa[sl] = jnp.ones((L,), jnp.float32) / jnp.maximum(
            acc, jnp.ones((L,), jnp.float32))
    pltpu.sync_copy(sa, inv_hbm.at[pl.ds(seg_base, SEG_SLICE)])


@functools.cache
def _build_sc_conserve():
    mesh = plsc.VectorSubcoreMesh(
        core_axis_name="c", subcore_axis_name="s", num_cores=1
    )
    return functools.partial(
        pl.kernel,
        out_type=jax.ShapeDtypeStruct((N,), jnp.float32),
        mesh=mesh,
        scratch_types=[
            pltpu.VMEM((CHUNK,), jnp.int32),
            pltpu.VMEM((CHUNK,), jnp.float32),
            pltpu.VMEM((CHUNK,), jnp.float32),
            pltpu.VMEM((CHUNK,), jnp.int32),
            pltpu.VMEM((NUM_SEG,), jnp.float32),
            pltpu.VMEM((NTILES, SEG_SLICE), jnp.float32),
            pltpu.VMEM((SEG_SLICE,), jnp.float32),
            pltpu.VMEM((SEG_SLICE,), jnp.float32),
            pltpu.VMEM_SHARED((NTILES * NUM_SEG,), jnp.float32),
            pltpu.VMEM_SHARED((NUM_SEG,), jnp.float32),
            pltpu.SemaphoreType.DMA,
            pltpu.SemaphoreType.DMA,
        ],
        compiler_params=pltpu.CompilerParams(needs_layout_passes=False),
    )(_sc_conserve_body)


def _sc_conserve_body(batch_hbm, charges_hbm, inv_hbm, out_hbm,
                      bvm, cvm, ovm, ivm, tvm, rvm, sa, sb,
                      tab_sh, delta_sh, sem0, sem1):
    sid = lax.axis_index("s")
    base = sid * CHUNK
    seg_base = sid * SEG_SLICE
    is_tail = sid == NTILES - 1

    @pl.when(jnp.logical_not(is_tail))
    def _full_loads():
        pltpu.async_copy(batch_hbm.at[pl.ds(base, CHUNK)], bvm, sem0)
        pltpu.async_copy(charges_hbm.at[pl.ds(base, CHUNK)], cvm, sem1)

    @pl.when(is_tail)
    def _tail_loads():
        pltpu.async_copy(batch_hbm.at[pl.ds(base, TAIL)],
                         bvm.at[pl.ds(0, TAIL)], sem0)
        pltpu.async_copy(charges_hbm.at[pl.ds(base, TAIL)],
                         cvm.at[pl.ds(0, TAIL)], sem1)

    def _zbody(i, _):
        tvm[pl.ds(i * L, L)] = jnp.zeros((L,), jnp.float32)
        return 0
    lax.fori_loop(0, NUM_SEG // L, _zbody, 0)
    pltpu.sync_copy(inv_hbm.at[pl.ds(seg_base, SEG_SLICE)], sb)

    @pl.when(jnp.logical_not(is_tail))
    def _full_wait():
        pltpu.make_async_copy(batch_hbm.at[pl.ds(base, CHUNK)], bvm, sem0).wait()
        pltpu.make_async_copy(charges_hbm.at[pl.ds(base, CHUNK)], cvm, sem1).wait()

    @pl.when(is_tail)
    def _tail_wait():
        pltpu.make_async_copy(batch_hbm.at[pl.ds(base, TAIL)],
                              bvm.at[pl.ds(0, TAIL)], sem0).wait()
        pltpu.make_async_copy(charges_hbm.at[pl.ds(base, TAIL)],
                              cvm.at[pl.ds(0, TAIL)], sem1).wait()
        def _fbody(i, _):
            sl = pl.ds(TAIL + i * L, L)
            bvm[sl] = jnp.zeros((L,), jnp.int32)
            cvm[sl] = jnp.zeros((L,), jnp.float32)
            return 0
        lax.fori_loop(0, (CHUNK - TAIL) // L, _fbody, 0)

    pltpu.sync_copy(tvm, tab_sh.at[pl.ds(sid * NUM_SEG, NUM_SEG)])

    def _obody(i, _):
        sl = pl.ds(i * L, L)
        ivm[sl] = bvm[sl] + sid * NUM_SEG
        return 0
    lax.fori_loop(0, CHUNK // L, _obody, 0)
    pltpu.sync_copy(cvm, tab_sh.at[ivm], add=True)
    plsc.subcore_barrier()

    for t in range(NTILES):
        pltpu.sync_copy(
            tab_sh.at[pl.ds(t * NUM_SEG + seg_base, SEG_SLICE)], rvm.at[t])

    for s in range(SEG_SLICE // L):
        sl = pl.ds(s * L, L)
        acc = rvm[0, sl]
        for t in range(1, NTILES):
            acc = acc + rvm[t, sl]
        sa[sl] = (jnp.zeros((L,), jnp.float32) - acc) * sb[sl]
    pltpu.sync_copy(sa, delta_sh.at[pl.ds(seg_base, SEG_SLICE)])
    plsc.subcore_barrier()

    pltpu.sync_copy(delta_sh, tvm)

    def _gbody(i, _):
        sl = pl.ds(i * L, L)
        idx = bvm[sl]
        ovm[sl] = cvm[sl] + plsc.load_gather(tvm, [idx])
        return 0
    lax.fori_loop(0, CHUNK // L, _gbody, 0)

    @pl.when(jnp.logical_not(is_tail))
    def _full_store():
        pltpu.sync_copy(ovm, out_hbm.at[pl.ds(base, CHUNK)])

    @pl.when(is_tail)
    def _tail_store():
        pltpu.sync_copy(ovm.at[pl.ds(0, TAIL)], out_hbm.at[pl.ds(base, TAIL)])


def kernel(node_invariant, batch, W1, b1, W2, b2):
    mask = (jnp.arange(NTILES * CHUNK, dtype=jnp.int32) < N).astype(jnp.float32)
    inv = _build_sc_counts()(batch, mask)
    charges = _mlp(node_invariant, W1, b1, W2, b2)
    return _build_sc_conserve()(batch, charges, inv)
